# baseline (speedup 1.0000x reference)
import functools

import jax
import jax.numpy as jnp
import numpy as np
from jax.experimental import pallas as pl
from jax.experimental.pallas import tpu as pltpu

_NPOINTS = [2048, 512, 256, 128]
_RADII = [0.5, 1.0, 2.0, 4.0]
_NSAMPLES = [32, 32, 32, 32]


def _sq_dist(a, b):
    return jnp.sum(a * a, -1)[:, None] + jnp.sum(b * b, -1)[None, :] - 2.0 * (a @ b.T)


def _fps(xyz, npoint):
    N = xyz.shape[0]

    def body(i, state):
        idxs, dists, far = state
        idxs = idxs.at[i].set(far)
        c = xyz[far]
        d = jnp.sum((xyz - c) ** 2, -1)
        dists = jnp.minimum(dists, d)
        far = jnp.argmax(dists).astype(jnp.int32)
        return idxs, dists, far

    init = (jnp.zeros(npoint, jnp.int32), jnp.full(N, 1e10, jnp.float32), jnp.int32(0))
    idxs, _, _ = jax.lax.fori_loop(0, npoint, body, init)
    return idxs


def _ball_query(radius, nsample, xyz, new_xyz):
    N = xyz.shape[0]
    d2 = _sq_dist(new_xyz, xyz)
    within = d2 < radius * radius
    idx = jnp.where(within, jnp.arange(N)[None, :], N)
    idx = jnp.sort(idx, axis=1)[:, :nsample]
    nearest = jnp.argmin(d2, axis=1, keepdims=True)
    first = jnp.where(idx[:, :1] == N, nearest, idx[:, :1])
    return jnp.where(idx == N, first, idx)


def _mlp_pool_body(*refs, nw):
    g_ref = refs[0]
    wb = refs[1 : 1 + 2 * nw]
    out_ref = refs[1 + 2 * nw]
    s = pl.program_id(0)
    x = g_ref[0]
    for i in range(nw):
        w = wb[2 * i][...]
        b = wb[2 * i + 1][...]
        x = jax.nn.relu(jnp.dot(x, w, preferred_element_type=jnp.float32) + b)

    @pl.when(s == 0)
    def _():
        out_ref[...] = x

    @pl.when(s != 0)
    def _():
        out_ref[...] = jnp.maximum(out_ref[...], x)


def _mlp_pool(g_t, ws, bs):
    S, Q, C = g_t.shape
    nw = len(ws)
    cout = ws[-1].shape[1]
    bs2 = [b.reshape(1, -1) for b in bs]
    wb_specs = []
    for w, b in zip(ws, bs2):
        wb_specs.append(pl.BlockSpec(w.shape, lambda s: (0, 0)))
        wb_specs.append(pl.BlockSpec(b.shape, lambda s: (0, 0)))
    args = []
    for w, b in zip(ws, bs2):
        args.extend([w, b])
    return pl.pallas_call(
        functools.partial(_mlp_pool_body, nw=nw),
        grid=(S,),
        in_specs=[pl.BlockSpec((1, Q, C), lambda s: (s, 0, 0))] + wb_specs,
        out_specs=pl.BlockSpec((Q, cout), lambda s: (0, 0)),
        out_shape=jax.ShapeDtypeStruct((Q, cout), jnp.float32),
    )(g_t, *args)


def _mlp_body(*refs, nw):
    x_ref = refs[0]
    wb = refs[1 : 1 + 2 * nw]
    out_ref = refs[1 + 2 * nw]
    x = x_ref[...]
    for i in range(nw):
        w = wb[2 * i][...]
        b = wb[2 * i + 1][...]
        x = jax.nn.relu(jnp.dot(x, w, preferred_element_type=jnp.float32) + b)
    out_ref[...] = x


def _mlp(x, ws, bs):
    nw = len(ws)
    cout = ws[-1].shape[1]
    bs2 = [b.reshape(1, -1) for b in bs]
    args = []
    for w, b in zip(ws, bs2):
        args.extend([x for x in (w, b)])
    return pl.pallas_call(
        functools.partial(_mlp_body, nw=nw),
        out_shape=jax.ShapeDtypeStruct((x.shape[0], cout), jnp.float32),
    )(x, *args)


def _sa_layer(xyz, feats, npoint, radius, nsample, ws, bs):
    sidx = _fps(xyz, npoint)
    new_xyz = xyz[sidx]
    gidx = _ball_query(radius, nsample, xyz, new_xyz)
    g_xyz = xyz[gidx] - new_xyz[:, None, :]
    g = jnp.concatenate([g_xyz, feats[gidx]], -1)
    g_t = jnp.transpose(g, (1, 0, 2))
    pooled = _mlp_pool(g_t, ws, bs)
    return new_xyz, pooled, sidx


def _three_interp(unknown, known, known_feats):
    d2 = _sq_dist(unknown, known)
    neg, idx = jax.lax.top_k(-d2, 3)
    w = 1.0 / (-neg + 1e-8)
    w = w / jnp.sum(w, axis=1, keepdims=True)
    return jnp.sum(known_feats[idx] * w[..., None], axis=1)


def _fp_module(unknown, known, unknown_feats, known_feats, ws, bs):
    interp = _three_interp(unknown, known, known_feats)
    return _mlp(jnp.concatenate([interp, unknown_feats], -1), ws, bs)


def _single(points, params):
    xyz = points[:, :3]
    feats = points[:, 3:]
    e_xyz, e_f, sa_idx = [xyz], [feats], []
    for i in range(4):
        lx, lf, si = _sa_layer(
            e_xyz[i], e_f[i], _NPOINTS[i], _RADII[i], _NSAMPLES[i],
            params['sa%d_w' % i], params['sa%d_b' % i])
        e_xyz.append(lx)
        e_f.append(lf)
        sa_idx.append(si)
    f1 = _fp_module(e_xyz[3], e_xyz[4], e_f[3], e_f[4], params['fp1_w'], params['fp1_b'])
    f2 = _fp_module(e_xyz[2], e_xyz[3], e_f[2], f1, params['fp2_w'], params['fp2_b'])
    num_seed = e_xyz[2].shape[0]
    return f2.T, e_xyz[2], sa_idx[0][:num_seed]


def kernel(points, params):
    return jax.vmap(_single, in_axes=(0, None))(points, params)


# speedup vs baseline: 4.4769x; 4.4769x over previous
import functools

import jax
import jax.numpy as jnp
from jax.experimental import pallas as pl
from jax.experimental.pallas import tpu as pltpu

_NPOINTS = [2048, 512, 256, 128]
_RADII = [0.5, 1.0, 2.0, 4.0]
_NSAMPLE = 32


def _fps_run(x, y, z, npoint, out_rows):
    rows = x.shape[0]
    n = rows * 128
    iota_in = (jax.lax.broadcasted_iota(jnp.int32, (rows, 128), 0) * 128
               + jax.lax.broadcasted_iota(jnp.int32, (rows, 128), 1))
    iota_out = (jax.lax.broadcasted_iota(jnp.int32, (out_rows, 128), 0) * 128
                + jax.lax.broadcasted_iota(jnp.int32, (out_rows, 128), 1))

    def body(i, st):
        far, dists, selb, sx, sy, sz = st
        fm = iota_in == far
        cx = jnp.sum(jnp.where(fm, x, 0.0))
        cy = jnp.sum(jnp.where(fm, y, 0.0))
        cz = jnp.sum(jnp.where(fm, z, 0.0))
        om = iota_out == i
        selb = jnp.where(om, far, selb)
        sx = jnp.where(om, cx, sx)
        sy = jnp.where(om, cy, sy)
        sz = jnp.where(om, cz, sz)
        dx = x - cx
        dy = y - cy
        dz = z - cz
        d = (dx * dx + dz * dz) + dy * dy
        dists = jnp.minimum(dists, d)
        m = jnp.max(dists)
        far = jnp.min(jnp.where(dists == m, iota_in, n)).astype(jnp.int32)
        return far, dists, selb, sx, sy, sz

    init = (jnp.int32(0), jnp.full((rows, 128), 1e10, jnp.float32),
            jnp.zeros((out_rows, 128), jnp.int32),
            jnp.zeros((out_rows, 128), jnp.float32),
            jnp.zeros((out_rows, 128), jnp.float32),
            jnp.zeros((out_rows, 128), jnp.float32))
    _, _, selb, sx, sy, sz = jax.lax.fori_loop(0, npoint, body, init)
    return selb, sx, sy, sz


def _fps_kernel_body(x_ref, y_ref, z_ref, sidx0_ref,
                     c1x, c1y, c1z, c2x, c2y, c2z,
                     c3x, c3y, c3z, c4x, c4y, c4z):
    for b in range(2):
        s0, x1, y1, z1 = _fps_run(x_ref[b], y_ref[b], z_ref[b], 2048, 16)
        sidx0_ref[b] = s0
        c1x[b], c1y[b], c1z[b] = x1, y1, z1
        _, x2, y2, z2 = _fps_run(x1, y1, z1, 512, 4)
        c2x[b], c2y[b], c2z[b] = x2, y2, z2
        _, x3, y3, z3 = _fps_run(x2, y2, z2, 256, 2)
        c3x[b], c3y[b], c3z[b] = x3, y3, z3
        _, x4, y4, z4 = _fps_run(x3, y3, z3, 128, 1)
        c4x[b], c4y[b], c4z[b] = x4, y4, z4


def _fps_all(points):
    xs = points[:, :, 0].reshape(2, 64, 128)
    ys = points[:, :, 1].reshape(2, 64, 128)
    zs = points[:, :, 2].reshape(2, 64, 128)
    f32 = jnp.float32
    shapes = [jax.ShapeDtypeStruct((2, 16, 128), jnp.int32)]
    for r in (16, 4, 2, 1):
        for _ in range(3):
            shapes.append(jax.ShapeDtypeStruct((2, r, 128), f32))
    outs = pl.pallas_call(
        _fps_kernel_body,
        out_shape=tuple(shapes),
    )(xs, ys, zs)
    sidx0 = outs[0].reshape(2, 2048)
    coords = []
    for li, r in enumerate((16, 4, 2, 1)):
        cx, cy, cz = outs[1 + 3 * li: 4 + 3 * li]
        npt = r * 128
        coords.append(jnp.stack(
            [cx.reshape(2, npt), cy.reshape(2, npt), cz.reshape(2, npt)], -1))
    return sidx0, coords


def _sa_body(qpad_ref, kt_ref, ptcat_ref, *wb_out, nw, r2, nsample):
    ws = [wb_out[2 * i] for i in range(nw)]
    bs = [wb_out[2 * i + 1] for i in range(nw)]
    out_ref = wb_out[2 * nw]
    x_scr = wb_out[2 * nw + 1]
    qpad = qpad_ref[0]
    bq = qpad.shape[0]
    q3 = qpad[:, :3]
    kt = kt_ref[0]
    n = kt.shape[1]
    ptcat = ptcat_ref[0]
    kx, ky, kz = kt[0:1, :], kt[1:2, :], kt[2:3, :]
    kn = (kx * kx + kz * kz) + ky * ky
    q0, q1, q2 = qpad[:, 0:1], qpad[:, 1:2], qpad[:, 2:3]
    qn = (q0 * q0 + q2 * q2) + q1 * q1
    k3 = ptcat[:, :3]
    mm = jax.lax.dot_general(q3, k3, (((1,), (1,)), ((), ())),
                             preferred_element_type=jnp.float32)
    d2 = qn + kn - 2.0 * mm
    iota_n = jax.lax.broadcasted_iota(jnp.int32, (bq, n), 1)
    keys = jnp.where(d2 < r2, iota_n, n)
    mind2 = jnp.min(d2, axis=1, keepdims=True)
    nidx = jnp.min(jnp.where(d2 == mind2, iota_n, n), axis=1, keepdims=True)
    oh = (iota_n == nidx).astype(jnp.float32)
    near_row = jnp.dot(oh, ptcat, preferred_element_type=jnp.float32,
                       precision=jax.lax.Precision.HIGHEST)
    m0 = jnp.min(keys, axis=1, keepdims=True)
    oh0 = jnp.where((keys == m0) & (m0 < n), 1.0, 0.0)
    row0 = jnp.dot(oh0, ptcat, preferred_element_type=jnp.float32,
                       precision=jax.lax.Precision.HIGHEST)
    fr = jnp.where(m0 < n, row0, near_row)
    x_scr[0] = fr
    keys = jnp.where(keys == m0, n, keys)

    def sel_body(k, keys):
        m = jnp.min(keys, axis=1, keepdims=True)
        ohk = jnp.where((keys == m) & (m < n), 1.0, 0.0)
        rowk = jnp.dot(ohk, ptcat, preferred_element_type=jnp.float32,
                       precision=jax.lax.Precision.HIGHEST)
        rowk = jnp.where(m < n, rowk, fr)
        x_scr[pl.ds(k, 1)] = rowk[None]
        return jnp.where(keys == m, n, keys)

    jax.lax.fori_loop(1, nsample, sel_body, keys)
    xg = x_scr[...] - qpad[None]
    h = xg.reshape(nsample * bq, qpad.shape[1])
    for i in range(nw):
        h = jax.nn.relu(
            jnp.dot(h, ws[i][...], preferred_element_type=jnp.float32)
            + bs[i][...])
    h = h.reshape(nsample, bq, h.shape[1])
    out_ref[0] = jnp.max(h, axis=0)


def _sa_fused(qpad, kt, ptcat, ws, bs, radius, bq):
    B, Q, Cp = qpad.shape
    N = kt.shape[2]
    nw = len(ws)
    cout = ws[-1].shape[1]
    bs2 = [b.reshape(1, -1) for b in bs]
    wb = []
    wb_specs = []
    for w, b in zip(ws, bs2):
        wb.extend([w, b])
        wb_specs.append(pl.BlockSpec(w.shape, lambda bb, qq: (0, 0)))
        wb_specs.append(pl.BlockSpec((1, b.shape[1]), lambda bb, qq: (0, 0)))
    grid = (B, Q // bq)
    return pl.pallas_call(
        functools.partial(_sa_body, nw=nw, r2=radius * radius,
                          nsample=_NSAMPLE),
        grid=grid,
        in_specs=[
            pl.BlockSpec((1, bq, Cp), lambda bb, qq: (bb, qq, 0)),
            pl.BlockSpec((1, 3, N), lambda bb, qq: (bb, 0, 0)),
            pl.BlockSpec((1, N, Cp), lambda bb, qq: (bb, 0, 0)),
        ] + wb_specs,
        out_specs=pl.BlockSpec((1, bq, cout), lambda bb, qq: (bb, qq, 0)),
        out_shape=jax.ShapeDtypeStruct((B, Q, cout), jnp.float32),
        scratch_shapes=[pltpu.VMEM((_NSAMPLE, bq, Cp), jnp.float32)],
    )(qpad, kt, ptcat, *wb)


def _fp_body(u3_ref, kt_ref, k3_ref, kf_ref, uf_ref, *wb_out, nw):
    ws = [wb_out[2 * i] for i in range(nw)]
    bs = [wb_out[2 * i + 1] for i in range(nw)]
    out_ref = wb_out[2 * nw]
    u3 = u3_ref[0]
    kt = kt_ref[0]
    k3 = k3_ref[0]
    kf = kf_ref[0]
    uf = uf_ref[0]
    nk = kt.shape[1]
    bq = u3.shape[0]
    kx, ky, kz = kt[0:1, :], kt[1:2, :], kt[2:3, :]
    kn = (kx * kx + kz * kz) + ky * ky
    q0, q1, q2 = u3[:, 0:1], u3[:, 1:2], u3[:, 2:3]
    qn = (q0 * q0 + q2 * q2) + q1 * q1
    mm = jax.lax.dot_general(u3, k3, (((1,), (1,)), ((), ())),
                             preferred_element_type=jnp.float32)
    d2 = qn + kn - 2.0 * mm
    iota_n = jax.lax.broadcasted_iota(jnp.int32, (bq, nk), 1)
    fs, wsum = [], None
    wk_list = []
    for _ in range(3):
        m = jnp.min(d2, axis=1, keepdims=True)
        idx = jnp.min(jnp.where(d2 == m, iota_n, nk), axis=1, keepdims=True)
        oh = (iota_n == idx).astype(jnp.float32)
        fs.append(jnp.dot(oh, kf, preferred_element_type=jnp.float32,
                       precision=jax.lax.Precision.HIGHEST))
        wk = 1.0 / (m + 1e-8)
        wk_list.append(wk)
        d2 = jnp.where(iota_n == idx, jnp.float32(3.4e38), d2)
    wsum = (wk_list[0] + wk_list[2]) + wk_list[1]
    interp = (fs[0] * (wk_list[0] / wsum) + fs[2] * (wk_list[2] / wsum)) \
        + fs[1] * (wk_list[1] / wsum)
    h = jnp.concatenate([interp, uf], axis=1)
    for i in range(nw):
        h = jax.nn.relu(
            jnp.dot(h, ws[i][...], preferred_element_type=jnp.float32)
            + bs[i][...])
    out_ref[0] = h


def _fp_fused(u3, kt, k3, kf, uf, ws, bs):
    B, Q, _ = u3.shape
    NK = kt.shape[2]
    CF = kf.shape[2]
    CU = uf.shape[2]
    nw = len(ws)
    cout = ws[-1].shape[1]
    bs2 = [b.reshape(1, -1) for b in bs]
    wb = []
    wb_specs = []
    for w, b in zip(ws, bs2):
        wb.extend([w, b])
        wb_specs.append(pl.BlockSpec(w.shape, lambda bb: (0, 0)))
        wb_specs.append(pl.BlockSpec((1, b.shape[1]), lambda bb: (0, 0)))
    return pl.pallas_call(
        functools.partial(_fp_body, nw=nw),
        grid=(B,),
        in_specs=[
            pl.BlockSpec((1, Q, 3), lambda bb: (bb, 0, 0)),
            pl.BlockSpec((1, 3, NK), lambda bb: (bb, 0, 0)),
            pl.BlockSpec((1, NK, 3), lambda bb: (bb, 0, 0)),
            pl.BlockSpec((1, NK, CF), lambda bb: (bb, 0, 0)),
            pl.BlockSpec((1, Q, CU), lambda bb: (bb, 0, 0)),
        ] + wb_specs,
        out_specs=pl.BlockSpec((1, Q, cout), lambda bb: (bb, 0, 0)),
        out_shape=jax.ShapeDtypeStruct((B, Q, cout), jnp.float32),
    )(u3, kt, k3, kf, uf, *wb)


def kernel(points, params):
    B = points.shape[0]
    sidx0, coords = _fps_all(points)
    e_xyz = [points[:, :, :3]] + coords
    feats0 = points[:, :, 3:]
    e_f = [feats0]
    for li in range(4):
        src_xyz = e_xyz[li]
        src_f = e_f[li]
        q = e_xyz[li + 1]
        Cp = 3 + src_f.shape[2]
        qpad = jnp.concatenate(
            [q, jnp.zeros((B, q.shape[1], src_f.shape[2]), jnp.float32)], -1)
        kt = jnp.transpose(src_xyz, (0, 2, 1))
        ptcat = jnp.concatenate([src_xyz, src_f], -1)
        bq = 128 if li == 0 else q.shape[1]
        pooled = _sa_fused(qpad, kt, ptcat,
                           params['sa%d_w' % li], params['sa%d_b' % li],
                           _RADII[li], bq)
        e_f.append(pooled)
    kt4 = jnp.transpose(e_xyz[4], (0, 2, 1))
    f1 = _fp_fused(e_xyz[3], kt4, e_xyz[4], e_f[4], e_f[3],
                   params['fp1_w'], params['fp1_b'])
    kt3 = jnp.transpose(e_xyz[3], (0, 2, 1))
    f2 = _fp_fused(e_xyz[2], kt3, e_xyz[3], f1, e_f[2],
                   params['fp2_w'], params['fp2_b'])
    return (jnp.transpose(f2, (0, 2, 1)), e_xyz[2], sidx0[:, :512])


# speedup vs baseline: 5.5992x; 1.2507x over previous
import functools

import jax
import jax.numpy as jnp
from jax.experimental import pallas as pl
from jax.experimental.pallas import tpu as pltpu

_NPOINTS = [2048, 512, 256, 128]
_RADII = [0.5, 1.0, 2.0, 4.0]
_NSAMPLE = 32


def _fps_run2(pts, npoint, out_rows):
    rows = pts[0][0].shape[0]
    n = rows * 128
    iota_in = (jax.lax.broadcasted_iota(jnp.int32, (rows, 128), 0) * 128
               + jax.lax.broadcasted_iota(jnp.int32, (rows, 128), 1))
    iota_out = (jax.lax.broadcasted_iota(jnp.int32, (out_rows, 128), 0) * 128
                + jax.lax.broadcasted_iota(jnp.int32, (out_rows, 128), 1))

    def body(i, st):
        om = iota_out == i
        new_st = []
        for b in range(2):
            far, dists, selb, sx, sy, sz = st[b]
            x, y, z = pts[b]
            fm = iota_in == far
            cx = jnp.sum(jnp.where(fm, x, 0.0))
            cy = jnp.sum(jnp.where(fm, y, 0.0))
            cz = jnp.sum(jnp.where(fm, z, 0.0))
            selb = jnp.where(om, far, selb)
            sx = jnp.where(om, cx, sx)
            sy = jnp.where(om, cy, sy)
            sz = jnp.where(om, cz, sz)
            dx = x - cx
            dy = y - cy
            dz = z - cz
            d = (dx * dx + dz * dz) + dy * dy
            dists = jnp.minimum(dists, d)
            m = jnp.max(dists)
            far = jnp.min(jnp.where(dists == m, iota_in, n)).astype(jnp.int32)
            new_st.append((far, dists, selb, sx, sy, sz))
        return tuple(new_st)

    init1 = (jnp.int32(0), jnp.full((rows, 128), 1e10, jnp.float32),
             jnp.zeros((out_rows, 128), jnp.int32),
             jnp.zeros((out_rows, 128), jnp.float32),
             jnp.zeros((out_rows, 128), jnp.float32),
             jnp.zeros((out_rows, 128), jnp.float32))
    fin = jax.lax.fori_loop(0, npoint, body, (init1, init1))
    return [(s[2], s[3], s[4], s[5]) for s in fin]


def _fps_kernel_body(x_ref, y_ref, z_ref, sidx0_ref,
                     c1x, c1y, c1z, c2x, c2y, c2z,
                     c3x, c3y, c3z, c4x, c4y, c4z):
    l1 = _fps_run2([(x_ref[b], y_ref[b], z_ref[b]) for b in range(2)],
                   2048, 16)
    for b in range(2):
        sidx0_ref[b] = l1[b][0]
        c1x[b], c1y[b], c1z[b] = l1[b][1], l1[b][2], l1[b][3]
    l2 = _fps_run2([l1[b][1:] for b in range(2)], 512, 4)
    for b in range(2):
        c2x[b], c2y[b], c2z[b] = l2[b][1], l2[b][2], l2[b][3]
    l3 = _fps_run2([l2[b][1:] for b in range(2)], 256, 2)
    for b in range(2):
        c3x[b], c3y[b], c3z[b] = l3[b][1], l3[b][2], l3[b][3]
    l4 = _fps_run2([l3[b][1:] for b in range(2)], 128, 1)
    for b in range(2):
        c4x[b], c4y[b], c4z[b] = l4[b][1], l4[b][2], l4[b][3]


def _fps_all(points):
    xs = points[:, :, 0].reshape(2, 64, 128)
    ys = points[:, :, 1].reshape(2, 64, 128)
    zs = points[:, :, 2].reshape(2, 64, 128)
    f32 = jnp.float32
    shapes = [jax.ShapeDtypeStruct((2, 16, 128), jnp.int32)]
    for r in (16, 4, 2, 1):
        for _ in range(3):
            shapes.append(jax.ShapeDtypeStruct((2, r, 128), f32))
    outs = pl.pallas_call(
        _fps_kernel_body,
        out_shape=tuple(shapes),
    )(xs, ys, zs)
    sidx0 = outs[0].reshape(2, 2048)
    coords = []
    for li, r in enumerate((16, 4, 2, 1)):
        cx, cy, cz = outs[1 + 3 * li: 4 + 3 * li]
        npt = r * 128
        coords.append(jnp.stack(
            [cx.reshape(2, npt), cy.reshape(2, npt), cz.reshape(2, npt)], -1))
    return sidx0, coords


def _sa_body(qpad_ref, kt_ref, ptcat_ref, *wb_out, nw, r2, nsample):
    ws = [wb_out[2 * i] for i in range(nw)]
    bs = [wb_out[2 * i + 1] for i in range(nw)]
    out_ref = wb_out[2 * nw]
    x_scr = wb_out[2 * nw + 1]
    qpad = qpad_ref[0]
    bq = qpad.shape[0]
    q3 = qpad[:, :3]
    kt = kt_ref[0]
    n = kt.shape[1]
    ptcat = ptcat_ref[0]
    kx, ky, kz = kt[0:1, :], kt[1:2, :], kt[2:3, :]
    kn = (kx * kx + kz * kz) + ky * ky
    q0, q1, q2 = qpad[:, 0:1], qpad[:, 1:2], qpad[:, 2:3]
    qn = (q0 * q0 + q2 * q2) + q1 * q1
    k3 = ptcat[:, :3]
    mm = jax.lax.dot_general(q3, k3, (((1,), (1,)), ((), ())),
                             preferred_element_type=jnp.float32)
    d2 = qn + kn - 2.0 * mm
    iota_n = jax.lax.broadcasted_iota(jnp.int32, (bq, n), 1)
    maskf = jnp.where(d2 < r2, 1.0, 0.0)
    nch = n // 128
    m2 = maskf.reshape(bq * nch, 128)
    iu = jax.lax.broadcasted_iota(jnp.int32, (128, 128), 0)
    ju = jax.lax.broadcasted_iota(jnp.int32, (128, 128), 1)
    tri = jnp.where(iu <= ju, 1.0, 0.0)
    cs = jnp.dot(m2, tri, preferred_element_type=jnp.float32)
    csr = cs.reshape(bq, nch, 128)
    tot = csr[:, :, 127]
    iu2 = jax.lax.broadcasted_iota(jnp.int32, (nch, nch), 0)
    ju2 = jax.lax.broadcasted_iota(jnp.int32, (nch, nch), 1)
    tri2 = jnp.where(iu2 < ju2, 1.0, 0.0)
    exc = jnp.dot(tot, tri2, preferred_element_type=jnp.float32)
    rank = (csr + exc.reshape(bq, nch, 1)).reshape(bq, n)
    cnt = rank[:, n - 1:]
    slot = jnp.where(maskf > 0.0, jnp.minimum(rank - 1.0, 32.0), 32.0)
    mind2 = jnp.min(d2, axis=1, keepdims=True)
    nidx = jnp.min(jnp.where(d2 == mind2, iota_n, n), axis=1, keepdims=True)
    oh = (iota_n == nidx).astype(jnp.float32)
    near_row = jnp.dot(oh, ptcat, preferred_element_type=jnp.float32,
                       precision=jax.lax.Precision.HIGHEST)
    oh0 = jnp.where(slot == 0.0, 1.0, 0.0)
    row0 = jnp.dot(oh0, ptcat, preferred_element_type=jnp.float32,
                   precision=jax.lax.Precision.HIGHEST)
    fr = jnp.where(cnt > 0.0, row0, near_row)
    x_scr[0] = fr

    def sel_body(k, carry):
        ohk = jnp.where(slot == k.astype(jnp.float32), 1.0, 0.0)
        rowk = jnp.dot(ohk, ptcat, preferred_element_type=jnp.float32,
                       precision=jax.lax.Precision.HIGHEST)
        rowk = jnp.where(cnt > k.astype(jnp.float32), rowk, fr)
        x_scr[pl.ds(k, 1)] = rowk[None]
        return carry

    jax.lax.fori_loop(1, nsample, sel_body, 0)
    xg = x_scr[...] - qpad[None]
    h = xg.reshape(nsample * bq, qpad.shape[1])
    for i in range(nw):
        h = jax.nn.relu(
            jnp.dot(h, ws[i][...], preferred_element_type=jnp.float32)
            + bs[i][...])
    h = h.reshape(nsample, bq, h.shape[1])
    out_ref[0] = jnp.max(h, axis=0)


def _sa_fused(qpad, kt, ptcat, ws, bs, radius, bq):
    B, Q, Cp = qpad.shape
    N = kt.shape[2]
    nw = len(ws)
    cout = ws[-1].shape[1]
    bs2 = [b.reshape(1, -1) for b in bs]
    wb = []
    wb_specs = []
    for w, b in zip(ws, bs2):
        wb.extend([w, b])
        wb_specs.append(pl.BlockSpec(w.shape, lambda bb, qq: (0, 0)))
        wb_specs.append(pl.BlockSpec((1, b.shape[1]), lambda bb, qq: (0, 0)))
    grid = (B, Q // bq)
    return pl.pallas_call(
        functools.partial(_sa_body, nw=nw, r2=radius * radius,
                          nsample=_NSAMPLE),
        grid=grid,
        in_specs=[
            pl.BlockSpec((1, bq, Cp), lambda bb, qq: (bb, qq, 0)),
            pl.BlockSpec((1, 3, N), lambda bb, qq: (bb, 0, 0)),
            pl.BlockSpec((1, N, Cp), lambda bb, qq: (bb, 0, 0)),
        ] + wb_specs,
        out_specs=pl.BlockSpec((1, bq, cout), lambda bb, qq: (bb, qq, 0)),
        out_shape=jax.ShapeDtypeStruct((B, Q, cout), jnp.float32),
        scratch_shapes=[pltpu.VMEM((_NSAMPLE, bq, Cp), jnp.float32)],
    )(qpad, kt, ptcat, *wb)


def _fp_body(u3_ref, kt_ref, k3_ref, kf_ref, uf_ref, *wb_out, nw):
    ws = [wb_out[2 * i] for i in range(nw)]
    bs = [wb_out[2 * i + 1] for i in range(nw)]
    out_ref = wb_out[2 * nw]
    u3 = u3_ref[0]
    kt = kt_ref[0]
    k3 = k3_ref[0]
    kf = kf_ref[0]
    uf = uf_ref[0]
    nk = kt.shape[1]
    bq = u3.shape[0]
    kx, ky, kz = kt[0:1, :], kt[1:2, :], kt[2:3, :]
    kn = (kx * kx + kz * kz) + ky * ky
    q0, q1, q2 = u3[:, 0:1], u3[:, 1:2], u3[:, 2:3]
    qn = (q0 * q0 + q2 * q2) + q1 * q1
    mm = jax.lax.dot_general(u3, k3, (((1,), (1,)), ((), ())),
                             preferred_element_type=jnp.float32)
    d2 = qn + kn - 2.0 * mm
    iota_n = jax.lax.broadcasted_iota(jnp.int32, (bq, nk), 1)
    fs, wsum = [], None
    wk_list = []
    for _ in range(3):
        m = jnp.min(d2, axis=1, keepdims=True)
        idx = jnp.min(jnp.where(d2 == m, iota_n, nk), axis=1, keepdims=True)
        oh = (iota_n == idx).astype(jnp.float32)
        fs.append(jnp.dot(oh, kf, preferred_element_type=jnp.float32,
                       precision=jax.lax.Precision.HIGHEST))
        wk = 1.0 / (m + 1e-8)
        wk_list.append(wk)
        d2 = jnp.where(iota_n == idx, jnp.float32(3.4e38), d2)
    wsum = (wk_list[0] + wk_list[2]) + wk_list[1]
    interp = (fs[0] * (wk_list[0] / wsum) + fs[2] * (wk_list[2] / wsum)) \
        + fs[1] * (wk_list[1] / wsum)
    h = jnp.concatenate([interp, uf], axis=1)
    for i in range(nw):
        h = jax.nn.relu(
            jnp.dot(h, ws[i][...], preferred_element_type=jnp.float32)
            + bs[i][...])
    out_ref[0] = h


def _fp_fused(u3, kt, k3, kf, uf, ws, bs):
    B, Q, _ = u3.shape
    NK = kt.shape[2]
    CF = kf.shape[2]
    CU = uf.shape[2]
    nw = len(ws)
    cout = ws[-1].shape[1]
    bs2 = [b.reshape(1, -1) for b in bs]
    wb = []
    wb_specs = []
    for w, b in zip(ws, bs2):
        wb.extend([w, b])
        wb_specs.append(pl.BlockSpec(w.shape, lambda bb: (0, 0)))
        wb_specs.append(pl.BlockSpec((1, b.shape[1]), lambda bb: (0, 0)))
    return pl.pallas_call(
        functools.partial(_fp_body, nw=nw),
        grid=(B,),
        in_specs=[
            pl.BlockSpec((1, Q, 3), lambda bb: (bb, 0, 0)),
            pl.BlockSpec((1, 3, NK), lambda bb: (bb, 0, 0)),
            pl.BlockSpec((1, NK, 3), lambda bb: (bb, 0, 0)),
            pl.BlockSpec((1, NK, CF), lambda bb: (bb, 0, 0)),
            pl.BlockSpec((1, Q, CU), lambda bb: (bb, 0, 0)),
        ] + wb_specs,
        out_specs=pl.BlockSpec((1, Q, cout), lambda bb: (bb, 0, 0)),
        out_shape=jax.ShapeDtypeStruct((B, Q, cout), jnp.float32),
    )(u3, kt, k3, kf, uf, *wb)


def kernel(points, params):
    B = points.shape[0]
    sidx0, coords = _fps_all(points)
    e_xyz = [points[:, :, :3]] + coords
    feats0 = points[:, :, 3:]
    e_f = [feats0]
    for li in range(4):
        src_xyz = e_xyz[li]
        src_f = e_f[li]
        q = e_xyz[li + 1]
        Cp = 3 + src_f.shape[2]
        qpad = jnp.concatenate(
            [q, jnp.zeros((B, q.shape[1], src_f.shape[2]), jnp.float32)], -1)
        kt = jnp.transpose(src_xyz, (0, 2, 1))
        ptcat = jnp.concatenate([src_xyz, src_f], -1)
        bq = 128 if li == 0 else q.shape[1]
        pooled = _sa_fused(qpad, kt, ptcat,
                           params['sa%d_w' % li], params['sa%d_b' % li],
                           _RADII[li], bq)
        e_f.append(pooled)
    kt4 = jnp.transpose(e_xyz[4], (0, 2, 1))
    f1 = _fp_fused(e_xyz[3], kt4, e_xyz[4], e_f[4], e_f[3],
                   params['fp1_w'], params['fp1_b'])
    kt3 = jnp.transpose(e_xyz[3], (0, 2, 1))
    f2 = _fp_fused(e_xyz[2], kt3, e_xyz[3], f1, e_f[2],
                   params['fp2_w'], params['fp2_b'])
    return (jnp.transpose(f2, (0, 2, 1)), e_xyz[2], sidx0[:, :512])


# speedup vs baseline: 7.5769x; 1.3532x over previous
import functools

import jax
import jax.numpy as jnp
from jax.experimental import pallas as pl
from jax.experimental.pallas import tpu as pltpu

_NPOINTS = [2048, 512, 256, 128]
_RADII = [0.5, 1.0, 2.0, 4.0]
_NSAMPLE = 32


def _fps_run2(pts, npoint, out_rows):
    rows = pts[0][0].shape[0]
    n = rows * 128
    iota_in = (jax.lax.broadcasted_iota(jnp.int32, (rows, 128), 0) * 128
               + jax.lax.broadcasted_iota(jnp.int32, (rows, 128), 1))
    iota_out = (jax.lax.broadcasted_iota(jnp.int32, (out_rows, 128), 0) * 128
                + jax.lax.broadcasted_iota(jnp.int32, (out_rows, 128), 1))

    def body(i, st):
        om = iota_out == i
        new_st = []
        for b in range(2):
            far, dists, selb, sx, sy, sz = st[b]
            x, y, z = pts[b]
            fm = iota_in == far
            cx = jnp.sum(jnp.where(fm, x, 0.0))
            cy = jnp.sum(jnp.where(fm, y, 0.0))
            cz = jnp.sum(jnp.where(fm, z, 0.0))
            selb = jnp.where(om, far, selb)
            sx = jnp.where(om, cx, sx)
            sy = jnp.where(om, cy, sy)
            sz = jnp.where(om, cz, sz)
            dx = x - cx
            dy = y - cy
            dz = z - cz
            d = (dx * dx + dz * dz) + dy * dy
            dists = jnp.minimum(dists, d)
            m = jnp.max(dists)
            far = jnp.min(jnp.where(dists == m, iota_in, n)).astype(jnp.int32)
            new_st.append((far, dists, selb, sx, sy, sz))
        return tuple(new_st)

    init1 = (jnp.int32(0), jnp.full((rows, 128), 1e10, jnp.float32),
             jnp.zeros((out_rows, 128), jnp.int32),
             jnp.zeros((out_rows, 128), jnp.float32),
             jnp.zeros((out_rows, 128), jnp.float32),
             jnp.zeros((out_rows, 128), jnp.float32))
    fin = jax.lax.fori_loop(0, npoint, body, (init1, init1))
    return [(s[2], s[3], s[4], s[5]) for s in fin]


def _fps_kernel_body(x_ref, y_ref, z_ref, sidx0_ref,
                     c1x, c1y, c1z, c2x, c2y, c2z,
                     c3x, c3y, c3z, c4x, c4y, c4z):
    l1 = _fps_run2([(x_ref[b], y_ref[b], z_ref[b]) for b in range(2)],
                   2048, 16)
    for b in range(2):
        sidx0_ref[b] = l1[b][0]
        c1x[b], c1y[b], c1z[b] = l1[b][1], l1[b][2], l1[b][3]
    l2 = _fps_run2([l1[b][1:] for b in range(2)], 512, 4)
    for b in range(2):
        c2x[b], c2y[b], c2z[b] = l2[b][1], l2[b][2], l2[b][3]
    l3 = _fps_run2([l2[b][1:] for b in range(2)], 256, 2)
    for b in range(2):
        c3x[b], c3y[b], c3z[b] = l3[b][1], l3[b][2], l3[b][3]
    l4 = _fps_run2([l3[b][1:] for b in range(2)], 128, 1)
    for b in range(2):
        c4x[b], c4y[b], c4z[b] = l4[b][1], l4[b][2], l4[b][3]


def _fps_all(points):
    xs = points[:, :, 0].reshape(2, 64, 128)
    ys = points[:, :, 1].reshape(2, 64, 128)
    zs = points[:, :, 2].reshape(2, 64, 128)
    f32 = jnp.float32
    shapes = [jax.ShapeDtypeStruct((2, 16, 128), jnp.int32)]
    for r in (16, 4, 2, 1):
        for _ in range(3):
            shapes.append(jax.ShapeDtypeStruct((2, r, 128), f32))
    outs = pl.pallas_call(
        _fps_kernel_body,
        out_shape=tuple(shapes),
    )(xs, ys, zs)
    sidx0 = outs[0].reshape(2, 2048)
    coords = []
    for li, r in enumerate((16, 4, 2, 1)):
        cx, cy, cz = outs[1 + 3 * li: 4 + 3 * li]
        npt = r * 128
        coords.append(jnp.stack(
            [cx.reshape(2, npt), cy.reshape(2, npt), cz.reshape(2, npt)], -1))
    return sidx0, coords


def _sa_body(qpad_ref, kt_ref, ptcat_ref, *wb_out, nw, r2, nsample):
    ws = [wb_out[2 * i] for i in range(nw)]
    bs = [wb_out[2 * i + 1] for i in range(nw)]
    out_ref = wb_out[2 * nw]
    x_scr = wb_out[2 * nw + 1]
    qpad = qpad_ref[0]
    bq = qpad.shape[0]
    q3 = qpad[:, :3]
    kt = kt_ref[0]
    n = kt.shape[1]
    ptcat = ptcat_ref[0]
    kx, ky, kz = kt[0:1, :], kt[1:2, :], kt[2:3, :]
    kn = (kx * kx + kz * kz) + ky * ky
    q0, q1, q2 = qpad[:, 0:1], qpad[:, 1:2], qpad[:, 2:3]
    qn = (q0 * q0 + q2 * q2) + q1 * q1
    k3 = ptcat[:, :3]
    mm = jax.lax.dot_general(q3, k3, (((1,), (1,)), ((), ())),
                             preferred_element_type=jnp.float32)
    d2 = qn + kn - 2.0 * mm
    iota_n = jax.lax.broadcasted_iota(jnp.int32, (bq, n), 1)
    maskf = jnp.where(d2 < r2, 1.0, 0.0)
    nch = n // 128
    m2 = maskf.reshape(bq * nch, 128)
    iu = jax.lax.broadcasted_iota(jnp.int32, (128, 128), 0)
    ju = jax.lax.broadcasted_iota(jnp.int32, (128, 128), 1)
    tri = jnp.where(iu <= ju, 1.0, 0.0)
    cs = jnp.dot(m2, tri, preferred_element_type=jnp.float32)
    csr = cs.reshape(bq, nch, 128)
    tot = csr[:, :, 127]
    iu2 = jax.lax.broadcasted_iota(jnp.int32, (nch, nch), 0)
    ju2 = jax.lax.broadcasted_iota(jnp.int32, (nch, nch), 1)
    tri2 = jnp.where(iu2 < ju2, 1.0, 0.0)
    exc = jnp.dot(tot, tri2, preferred_element_type=jnp.float32)
    rank = (csr + exc.reshape(bq, nch, 1)).reshape(bq, n)
    cnt = rank[:, n - 1:]
    slot = jnp.where(maskf > 0.0, jnp.minimum(rank - 1.0, 32.0), 32.0)
    mind2 = jnp.min(d2, axis=1, keepdims=True)
    nidx = jnp.min(jnp.where(d2 == mind2, iota_n, n), axis=1, keepdims=True)
    pt_hi = ptcat.astype(jnp.bfloat16).astype(jnp.float32)
    r1 = ptcat - pt_hi
    pt_mid = r1.astype(jnp.bfloat16).astype(jnp.float32)
    pt_lo = r1 - pt_mid

    def gather_rows(ohk):
        g = jnp.dot(ohk, pt_hi, preferred_element_type=jnp.float32)
        g = g + jnp.dot(ohk, pt_mid, preferred_element_type=jnp.float32)
        return g + jnp.dot(ohk, pt_lo, preferred_element_type=jnp.float32)

    oh = (iota_n == nidx).astype(jnp.float32)
    near_row = gather_rows(oh)
    oh0 = jnp.where(slot == 0.0, 1.0, 0.0)
    row0 = gather_rows(oh0)
    fr = jnp.where(cnt > 0.0, row0, near_row)
    x_scr[0] = fr
    for k in range(1, nsample):
        kf = jnp.float32(k)
        ohk = jnp.where(slot == kf, 1.0, 0.0)
        rowk = gather_rows(ohk)
        rowk = jnp.where(cnt > kf, rowk, fr)
        x_scr[k] = rowk
    xg = x_scr[...] - qpad[None]
    h = xg.reshape(nsample * bq, qpad.shape[1])
    for i in range(nw):
        h = jax.nn.relu(
            jnp.dot(h, ws[i][...], preferred_element_type=jnp.float32)
            + bs[i][...])
    h = h.reshape(nsample, bq, h.shape[1])
    out_ref[0] = jnp.max(h, axis=0)


def _sa_fused(qpad, kt, ptcat, ws, bs, radius, bq):
    B, Q, Cp = qpad.shape
    N = kt.shape[2]
    nw = len(ws)
    cout = ws[-1].shape[1]
    bs2 = [b.reshape(1, -1) for b in bs]
    wb = []
    wb_specs = []
    for w, b in zip(ws, bs2):
        wb.extend([w, b])
        wb_specs.append(pl.BlockSpec(w.shape, lambda bb, qq: (0, 0)))
        wb_specs.append(pl.BlockSpec((1, b.shape[1]), lambda bb, qq: (0, 0)))
    grid = (B, Q // bq)
    return pl.pallas_call(
        functools.partial(_sa_body, nw=nw, r2=radius * radius,
                          nsample=_NSAMPLE),
        grid=grid,
        in_specs=[
            pl.BlockSpec((1, bq, Cp), lambda bb, qq: (bb, qq, 0)),
            pl.BlockSpec((1, 3, N), lambda bb, qq: (bb, 0, 0)),
            pl.BlockSpec((1, N, Cp), lambda bb, qq: (bb, 0, 0)),
        ] + wb_specs,
        out_specs=pl.BlockSpec((1, bq, cout), lambda bb, qq: (bb, qq, 0)),
        out_shape=jax.ShapeDtypeStruct((B, Q, cout), jnp.float32),
        scratch_shapes=[pltpu.VMEM((_NSAMPLE, bq, Cp), jnp.float32)],
    )(qpad, kt, ptcat, *wb)


def _fp_body(u3_ref, kt_ref, k3_ref, kf_ref, uf_ref, *wb_out, nw):
    ws = [wb_out[2 * i] for i in range(nw)]
    bs = [wb_out[2 * i + 1] for i in range(nw)]
    out_ref = wb_out[2 * nw]
    u3 = u3_ref[0]
    kt = kt_ref[0]
    k3 = k3_ref[0]
    kf = kf_ref[0]
    uf = uf_ref[0]
    nk = kt.shape[1]
    bq = u3.shape[0]
    kx, ky, kz = kt[0:1, :], kt[1:2, :], kt[2:3, :]
    kn = (kx * kx + kz * kz) + ky * ky
    q0, q1, q2 = u3[:, 0:1], u3[:, 1:2], u3[:, 2:3]
    qn = (q0 * q0 + q2 * q2) + q1 * q1
    mm = jax.lax.dot_general(u3, k3, (((1,), (1,)), ((), ())),
                             preferred_element_type=jnp.float32)
    d2 = qn + kn - 2.0 * mm
    iota_n = jax.lax.broadcasted_iota(jnp.int32, (bq, nk), 1)
    fs, wsum = [], None
    wk_list = []
    for _ in range(3):
        m = jnp.min(d2, axis=1, keepdims=True)
        idx = jnp.min(jnp.where(d2 == m, iota_n, nk), axis=1, keepdims=True)
        oh = (iota_n == idx).astype(jnp.float32)
        fs.append(jnp.dot(oh, kf, preferred_element_type=jnp.float32,
                       precision=jax.lax.Precision.HIGHEST))
        wk = 1.0 / (m + 1e-8)
        wk_list.append(wk)
        d2 = jnp.where(iota_n == idx, jnp.float32(3.4e38), d2)
    wsum = (wk_list[0] + wk_list[2]) + wk_list[1]
    interp = (fs[0] * (wk_list[0] / wsum) + fs[2] * (wk_list[2] / wsum)) \
        + fs[1] * (wk_list[1] / wsum)
    h = jnp.concatenate([interp, uf], axis=1)
    for i in range(nw):
        h = jax.nn.relu(
            jnp.dot(h, ws[i][...], preferred_element_type=jnp.float32)
            + bs[i][...])
    out_ref[0] = h


def _fp_fused(u3, kt, k3, kf, uf, ws, bs):
    B, Q, _ = u3.shape
    NK = kt.shape[2]
    CF = kf.shape[2]
    CU = uf.shape[2]
    nw = len(ws)
    cout = ws[-1].shape[1]
    bs2 = [b.reshape(1, -1) for b in bs]
    wb = []
    wb_specs = []
    for w, b in zip(ws, bs2):
        wb.extend([w, b])
        wb_specs.append(pl.BlockSpec(w.shape, lambda bb: (0, 0)))
        wb_specs.append(pl.BlockSpec((1, b.shape[1]), lambda bb: (0, 0)))
    return pl.pallas_call(
        functools.partial(_fp_body, nw=nw),
        grid=(B,),
        in_specs=[
            pl.BlockSpec((1, Q, 3), lambda bb: (bb, 0, 0)),
            pl.BlockSpec((1, 3, NK), lambda bb: (bb, 0, 0)),
            pl.BlockSpec((1, NK, 3), lambda bb: (bb, 0, 0)),
            pl.BlockSpec((1, NK, CF), lambda bb: (bb, 0, 0)),
            pl.BlockSpec((1, Q, CU), lambda bb: (bb, 0, 0)),
        ] + wb_specs,
        out_specs=pl.BlockSpec((1, Q, cout), lambda bb: (bb, 0, 0)),
        out_shape=jax.ShapeDtypeStruct((B, Q, cout), jnp.float32),
    )(u3, kt, k3, kf, uf, *wb)


def kernel(points, params):
    B = points.shape[0]
    sidx0, coords = _fps_all(points)
    e_xyz = [points[:, :, :3]] + coords
    feats0 = points[:, :, 3:]
    e_f = [feats0]
    for li in range(4):
        src_xyz = e_xyz[li]
        src_f = e_f[li]
        q = e_xyz[li + 1]
        Cp = 3 + src_f.shape[2]
        qpad = jnp.concatenate(
            [q, jnp.zeros((B, q.shape[1], src_f.shape[2]), jnp.float32)], -1)
        kt = jnp.transpose(src_xyz, (0, 2, 1))
        ptcat = jnp.concatenate([src_xyz, src_f], -1)
        bq = 128 if li == 0 else q.shape[1]
        pooled = _sa_fused(qpad, kt, ptcat,
                           params['sa%d_w' % li], params['sa%d_b' % li],
                           _RADII[li], bq)
        e_f.append(pooled)
    kt4 = jnp.transpose(e_xyz[4], (0, 2, 1))
    f1 = _fp_fused(e_xyz[3], kt4, e_xyz[4], e_f[4], e_f[3],
                   params['fp1_w'], params['fp1_b'])
    kt3 = jnp.transpose(e_xyz[3], (0, 2, 1))
    f2 = _fp_fused(e_xyz[2], kt3, e_xyz[3], f1, e_f[2],
                   params['fp2_w'], params['fp2_b'])
    return (jnp.transpose(f2, (0, 2, 1)), e_xyz[2], sidx0[:, :512])
